# Initial kernel scaffold; baseline (speedup 1.0000x reference)
import jax
import jax.numpy as jnp
from jax.experimental import pallas as pl


def kernel(x, src_lens, duration_target, pitch_target, energy_target, max_len, W1s, b1s, g1s, be1s, W2s, b2s, g2s, be2s, Wls, bls, pitch_emb, energy_emb, pitch_bins, energy_bins):
    raise NotImplementedError("write your pallas kernel here")


# speedup vs baseline: 48.5336x; 48.5336x over previous
import functools

import jax
import jax.numpy as jnp
from jax import lax
from jax.experimental import pallas as pl
from jax.experimental.pallas import tpu as pltpu
from jax.experimental.pallas import tpu_sc as plsc

B, S, E = 16, 512, 256
FILT = 256
N_BINS = 256
MAXL = 2048
NC, NS = 2, 16
NW = NC * NS
FPW = (B * MAXL) // NW
CH = 128


def _tc_body(sl_ref, bl_ref,
             x_ref, d_ref, pt_ref, et_ref,
             w1_ref, b1_ref, g1_ref, be1_ref,
             w2_ref, b2_ref, g2_ref, be2_ref,
             wl_ref, pemb_ref, eemb_ref,
             blo_p_ref, bhi_p_ref, blo_e_ref, bhi_e_ref,
             logd_ref, pp_ref, ep_ref, x2_ref, gidx_ref, mlen_ref, mask_ref):
    b = pl.program_id(0)

    @pl.when(b == B)
    def _zero_block():
        x2_ref[...] = jnp.zeros((1, S, E), jnp.float32)

    @pl.when(b < B)
    def _main():
        x = x_ref[0]
        sl = sl_ref[b]
        tokc = lax.broadcasted_iota(jnp.int32, (S, 1), 0)
        padc = tokc >= sl

        zrow = jnp.zeros((1, E), jnp.float32)
        xcat = jnp.concatenate(
            [jnp.concatenate([zrow, x[:-1]], axis=0), x,
             jnp.concatenate([x[1:], zrow], axis=0)], axis=1)

        def ln(h, g, be):
            m = jnp.mean(h, axis=-1, keepdims=True)
            c = h - m
            v = jnp.mean(c * c, axis=-1, keepdims=True)
            return c * lax.rsqrt(v + 1e-5) * g[None, :] + be[None, :]

        def predictor(i, out_ref):
            h = jnp.maximum(xcat @ w1_ref[i] + b1_ref[i][None, :], 0.0)
            h = ln(h, g1_ref[i], be1_ref[i])
            zr = jnp.zeros((1, FILT), jnp.float32)
            hcat = jnp.concatenate(
                [jnp.concatenate([zr, h[:-1]], axis=0), h,
                 jnp.concatenate([h[1:], zr], axis=0)], axis=1)
            h2 = jnp.maximum(hcat @ w2_ref[i] + b2_ref[i][None, :], 0.0)
            h2 = ln(h2, g2_ref[i], be2_ref[i])
            o = jnp.sum(h2 * wl_ref[i], axis=1, keepdims=True) + bl_ref[i, 0]
            out_ref[0] = jnp.where(padc, 0.0, o)

        predictor(0, logd_ref)
        predictor(1, pp_ref)
        predictor(2, ep_ref)

        pt = pt_ref[0]
        oh_p = ((pt >= blo_p_ref[0][None, :]).astype(jnp.float32)
                - (pt >= bhi_p_ref[0][None, :]).astype(jnp.float32))
        et = et_ref[0]
        oh_e = ((et >= blo_e_ref[0][None, :]).astype(jnp.float32)
                - (et >= bhi_e_ref[0][None, :]).astype(jnp.float32))
        x2_ref[0] = x + oh_p @ pemb_ref[...] + oh_e @ eemb_ref[...]

        drow = d_ref[0].astype(jnp.float32)
        tokr = lax.broadcasted_iota(jnp.int32, (1, S), 1)
        dmask = jnp.where(tokr >= sl, 0.0, drow)
        ii = lax.broadcasted_iota(jnp.int32, (S, S), 0)
        jj = lax.broadcasted_iota(jnp.int32, (S, S), 1)
        cum = dmask @ (ii <= jj).astype(jnp.float32)

        total = jnp.sum(dmask).astype(jnp.int32)
        mlen = jnp.minimum(total, MAXL)
        mlen_ref[0] = jnp.full((1, 128), mlen, jnp.int32)

        frames = lax.broadcasted_iota(jnp.int32, (MAXL, 1), 0)
        gef = (cum <= frames.astype(jnp.float32)).astype(jnp.float32)
        idx = jnp.clip(jnp.sum(gef, axis=1, keepdims=True).astype(jnp.int32),
                       0, S - 1)
        zidx = B * S + (frames & (S - 1))
        gidx_ref[0] = jnp.where(frames < mlen, idx + b * S, zidx)
        mask_ref[0] = (frames >= mlen).astype(jnp.int32)


def _tc_call(src_lens, bls, x, dur3, pt3, et3, W1r, b1s, g1s, be1s,
             W2r, b2s, g2s, be2s, Wlr, pemb, eemb,
             blo_p, bhi_p, blo_e, bhi_e):
    out_shape = (
        jax.ShapeDtypeStruct((B, S, 1), jnp.float32),
        jax.ShapeDtypeStruct((B, S, 1), jnp.float32),
        jax.ShapeDtypeStruct((B, S, 1), jnp.float32),
        jax.ShapeDtypeStruct((B + 1, S, E), jnp.float32),
        jax.ShapeDtypeStruct((B, MAXL, 1), jnp.int32),
        jax.ShapeDtypeStruct((B, 1, 128), jnp.int32),
        jax.ShapeDtypeStruct((B, MAXL, 1), jnp.int32),
    )

    def full(shape):
        return pl.BlockSpec(shape, lambda b, n=len(shape): (0,) * n)

    def per_b(s1, s2):
        return pl.BlockSpec((1, s1, s2),
                            lambda b: (jnp.minimum(b, B - 1), 0, 0))

    return pl.pallas_call(
        _tc_body,
        grid=(B + 1,),
        in_specs=[
            pl.BlockSpec(memory_space=pltpu.SMEM),
            pl.BlockSpec(memory_space=pltpu.SMEM),
            per_b(S, E),
            per_b(1, S),
            per_b(S, 1),
            per_b(S, 1),
            full((3, 3 * E, FILT)),
            full((3, FILT)), full((3, FILT)), full((3, FILT)),
            full((3, 3 * FILT, FILT)),
            full((3, FILT)), full((3, FILT)), full((3, FILT)),
            full((3, 1, FILT)),
            full((N_BINS, E)), full((N_BINS, E)),
            full((1, N_BINS)), full((1, N_BINS)),
            full((1, N_BINS)), full((1, N_BINS)),
        ],
        out_specs=[
            per_b(S, 1), per_b(S, 1), per_b(S, 1),
            pl.BlockSpec((1, S, E), lambda b: (b, 0, 0)),
            per_b(MAXL, 1),
            per_b(1, 128),
            per_b(MAXL, 1),
        ],
        out_shape=out_shape,
    )(src_lens, bls, x, dur3, pt3, et3, W1r, b1s, g1s, be1s,
      W2r, b2s, g2s, be2s, Wlr, pemb, eemb, blo_p, bhi_p, blo_e, bhi_e)


def _sc_gather(x2f, gidxf):
    mesh = plsc.VectorSubcoreMesh(core_axis_name="c", subcore_axis_name="s")

    @functools.partial(
        pl.kernel,
        mesh=mesh,
        out_type=jax.ShapeDtypeStruct((B * MAXL, E), jnp.float32),
        scratch_types=[
            pltpu.VMEM((FPW,), jnp.int32),
            pltpu.VMEM((CH, E), jnp.float32),
            pltpu.VMEM((CH, E), jnp.float32),
            pltpu.SemaphoreType.DMA,
            pltpu.SemaphoreType.DMA,
        ],
    )
    def k(x2_hbm, gidx_hbm, out_hbm, idx_v, buf0, buf1, sem0, sem1):
        cid = lax.axis_index("c")
        sid = lax.axis_index("s")
        wid = sid * NC + cid
        base = wid * FPW
        pltpu.sync_copy(gidx_hbm.at[pl.ds(base, FPW)], idx_v)
        bufs = (buf0, buf1)
        sems = (sem0, sem1)
        nch = FPW // CH
        cps = [None] * nch
        cps[0] = pltpu.async_copy(x2_hbm.at[idx_v.at[pl.ds(0, CH)]],
                                  bufs[0], sems[0])
        for ci in range(nch):
            if ci + 1 < nch:
                cps[ci + 1] = pltpu.async_copy(
                    x2_hbm.at[idx_v.at[pl.ds((ci + 1) * CH, CH)]],
                    bufs[(ci + 1) % 2], sems[(ci + 1) % 2])
            cps[ci].wait()
            pltpu.sync_copy(bufs[ci % 2],
                            out_hbm.at[pl.ds(base + ci * CH, CH)])

    return k(x2f, gidxf)


def kernel(x, src_lens, duration_target, pitch_target, energy_target,
           max_len, W1s, b1s, g1s, be1s, W2s, b2s, g2s, be2s, Wls, bls,
           pitch_emb, energy_emb, pitch_bins, energy_bins):
    ninf = jnp.full((1,), -jnp.inf, jnp.float32)
    pinf = jnp.full((1,), jnp.inf, jnp.float32)
    blo_p = jnp.concatenate([ninf, pitch_bins]).reshape(1, N_BINS)
    bhi_p = jnp.concatenate([pitch_bins, pinf]).reshape(1, N_BINS)
    blo_e = jnp.concatenate([ninf, energy_bins]).reshape(1, N_BINS)
    bhi_e = jnp.concatenate([energy_bins, pinf]).reshape(1, N_BINS)

    logd3, pp3, ep3, x2p, gidx3, mlen3, mask3 = _tc_call(
        src_lens, bls, x,
        duration_target.reshape(B, 1, S),
        pitch_target.reshape(B, S, 1),
        energy_target.reshape(B, S, 1),
        W1s.reshape(3, 3 * E, FILT), b1s, g1s, be1s,
        W2s.reshape(3, 3 * FILT, FILT), b2s, g2s, be2s,
        Wls.reshape(3, 1, FILT), pitch_emb, energy_emb,
        blo_p, bhi_p, blo_e, bhi_e)

    mel = _sc_gather(x2p.reshape((B + 1) * S, E), gidx3.reshape(B * MAXL))
    return (mel.reshape(B, MAXL, E),
            logd3.reshape(B, S), pp3.reshape(B, S), ep3.reshape(B, S),
            mlen3[:, 0, 0], mask3.reshape(B, MAXL).astype(bool))
